# baseline (speedup 1.0000x reference)
import jax
import jax.numpy as jnp
from jax import lax
from jax.experimental import pallas as pl
from jax.experimental.pallas import tpu as pltpu
from jax.experimental.pallas import tpu_sc as plsc

_N = 7056
_L = 16
_NS = 16
_CHUNK = 448
_TAIL = _N - (_NS - 1) * _CHUNK
_NEG = float("-inf")
_BIG = jnp.int32(2**31 - 1)


def _sc_body(x_hbm, out_hbm, in_v, stg, pub, loc, out_v, zsem):
    s = lax.axis_index("s")
    base = s * _CHUNK
    last = s == _NS - 1

    zero = jnp.zeros((_L,), dtype=jnp.float32)
    for i in range(_CHUNK // _L):
        out_v[pl.ds(i * _L, _L)] = zero
    zcp_full = pltpu.make_async_copy(
        out_v, out_hbm.at[pl.ds(base, _CHUNK)], zsem)
    zcp_tail = pltpu.make_async_copy(
        out_v.at[pl.ds(0, _TAIL)], out_hbm.at[pl.ds(base, _TAIL)], zsem)

    @pl.when(~last)
    def _():
        zcp_full.start()

    @pl.when(last)
    def _():
        zcp_tail.start()

    @pl.when(~last)
    def _():
        pltpu.sync_copy(x_hbm.at[pl.ds(base, _CHUNK)], in_v)

    @pl.when(last)
    def _():
        pltpu.sync_copy(x_hbm.at[pl.ds((_NS - 1) * _CHUNK, _TAIL)],
                        in_v.at[pl.ds(0, _TAIL)])
        neg = jnp.full((_L,), _NEG, dtype=jnp.float32)
        for i in range(_TAIL // _L, _CHUNK // _L):
            in_v[pl.ds(i * _L, _L)] = neg

    lane = lax.iota(jnp.int32, _L)
    best_val = jnp.full((_L,), _NEG, dtype=jnp.float32)
    best_idx = jnp.zeros((_L,), dtype=jnp.int32)
    for i in range(_CHUNK // _L):
        v = in_v[pl.ds(i * _L, _L)]
        gidx = lane + (base + i * _L)
        take = v > best_val
        best_val = jnp.where(take, v, best_val)
        best_idx = jnp.where(take, gidx, best_idx)

    stg[0] = best_val
    stg[1] = plsc.bitcast(best_idx, jnp.float32)
    pltpu.sync_copy(stg, pub.at[s])
    plsc.subcore_barrier()
    pltpu.sync_copy(pub, loc)

    cur_val = jnp.full((_L,), _NEG, dtype=jnp.float32)
    cur_idx = jnp.full((_L,), _BIG, dtype=jnp.int32)
    for t in range(_NS):
        v = loc[t, 0]
        i = plsc.bitcast(loc[t, 1], jnp.int32)
        take = (v > cur_val) | ((v == cur_val) & (i < cur_idx))
        cur_val = jnp.where(take, v, cur_val)
        cur_idx = jnp.where(take, i, cur_idx)

    m = jnp.max(cur_val)
    cand = jnp.where(cur_val == m, cur_idx, _BIG)
    winner = jnp.min(cand)

    @pl.when(~last)
    def _():
        zcp_full.wait()

    @pl.when(last)
    def _():
        zcp_tail.wait()

    @pl.when(winner // _CHUNK == s)
    def _():
        vbase = pl.multiple_of((winner // _L) * _L, _L)
        stg[0] = ((lane + vbase) == winner).astype(jnp.float32)
        pltpu.sync_copy(stg.at[0], out_hbm.at[pl.ds(vbase, _L)])


def kernel(inputs):
    x = jnp.reshape(inputs, (_N,))
    sc_call = pl.kernel(
        _sc_body,
        out_type=jax.ShapeDtypeStruct((_N,), jnp.float32),
        mesh=plsc.VectorSubcoreMesh(core_axis_name="c", subcore_axis_name="s",
                                    num_cores=1, num_subcores=_NS),
        compiler_params=pltpu.CompilerParams(needs_layout_passes=False,
                                             use_tc_tiling_on_sc=False,
                                             skip_device_barrier=True),
        scratch_types=[
            pltpu.VMEM((_CHUNK,), jnp.float32),
            pltpu.VMEM((2, _L), jnp.float32),
            pltpu.VMEM_SHARED((_NS, 2, _L), jnp.float32),
            pltpu.VMEM((_NS, 2, _L), jnp.float32),
            pltpu.VMEM((_CHUNK,), jnp.float32),
            pltpu.SemaphoreType.DMA,
        ],
    )
    return jnp.reshape(sc_call(x), (1, _N))
